# baseline (speedup 1.0000x reference)
import functools

import jax
import jax.numpy as jnp
from jax import lax
from jax.experimental import pallas as pl
from jax.experimental.pallas import tpu as pltpu
from jax.experimental.pallas import tpu_sc as plsc

_NUM_CORES = 2
_NUM_SUBCORES = 16
_NUM_WORKERS = _NUM_CORES * _NUM_SUBCORES
_RPC = 4
_SLOTS = 3
_SENT = 2**30


def kernel(token_ids, weights):
    batch, seq = token_ids.shape
    num_idx = batch * seq
    num_rows, dim = weights.shape

    half = lax.shift_right_logical(token_ids, 1)
    halfp = jnp.pad(half, ((0, 0), (0, 128 - seq)), constant_values=_SENT)
    ih3 = halfp.reshape(batch // _RPC, _RPC, 128)
    wpair = weights.reshape(num_rows // 2, 2 * dim)

    rows_per_w = batch // _NUM_WORKERS
    n_chunks = rows_per_w // _RPC
    seqp = (seq + 7) // 8 * 8
    chunk_tok = _RPC * seqp
    buf_rows = (_RPC - 1) * seqp + 128

    mesh = plsc.VectorSubcoreMesh(core_axis_name="c", subcore_axis_name="s")

    scratch = (
        [pltpu.VMEM((buf_rows, 2 * dim), jnp.float32) for _ in range(_SLOTS)]
        + [pltpu.VMEM((_RPC, 128), jnp.int32) for _ in range(_SLOTS)]
        + [pltpu.SemaphoreType.DMA for _ in range(3 * _SLOTS)]
    )

    @functools.partial(
        pl.kernel,
        mesh=mesh,
        out_type=jax.ShapeDtypeStruct((batch * seqp, 2 * dim), weights.dtype),
        scratch_types=scratch,
    )
    def gather_kernel(table_hbm, ih_hbm, out_hbm, *scr):
        rvs = scr[0:_SLOTS]
        ivs = scr[_SLOTS : 2 * _SLOTS]
        isem = scr[2 * _SLOTS : 3 * _SLOTS]
        gsem = scr[3 * _SLOTS : 4 * _SLOTS]
        ssem = scr[4 * _SLOTS : 5 * _SLOTS]

        wid = lax.axis_index("s") * _NUM_CORES + lax.axis_index("c")
        chunk_base = wid * n_chunks

        def idx_copies(t, s):
            return (
                pltpu.make_async_copy(ih_hbm.at[chunk_base + t], ivs[s], isem[s]),
            )

        def gather_copies(t, s):
            return tuple(
                pltpu.make_async_copy(
                    table_hbm.at[
                        plsc.Indices(ivs[s].at[r], ignored_value=_SENT)
                    ],
                    rvs[s].at[pl.ds(r * seqp, 128)],
                    gsem[s],
                )
                for r in range(_RPC)
            )

        def store_copies(t, s):
            return (
                pltpu.make_async_copy(
                    rvs[s].at[pl.ds(0, chunk_tok)],
                    out_hbm.at[pl.ds((chunk_base + t) * chunk_tok, chunk_tok)],
                    ssem[s],
                ),
            )

        def start(cs):
            for c in cs:
                c.start()

        def wait(cs):
            for c in cs:
                c.wait()

        start(idx_copies(0, 0))
        for t in range(_SLOTS):
            s = t
            wait(idx_copies(t, s))
            start(gather_copies(t, s))
            start(idx_copies(t + 1, (t + 1) % _SLOTS))
            if t >= 1:
                wait(gather_copies(t - 1, s - 1))
                start(store_copies(t - 1, s - 1))

        @pl.loop(1, n_chunks // _SLOTS)
        def _(k):
            t0 = k * _SLOTS
            for j in range(_SLOTS):
                t = t0 + j
                s = j
                pj = (j - 1) % _SLOTS
                wait(idx_copies(t, s))
                wait(store_copies(t - _SLOTS, s))
                start(gather_copies(t, s))
                start(idx_copies(t + 1, (j + 1) % _SLOTS))
                wait(gather_copies(t - 1, pj))
                start(store_copies(t - 1, pj))

        for t in range(_SLOTS * (n_chunks // _SLOTS), n_chunks):
            s = t % _SLOTS
            pj = (s - 1) % _SLOTS
            wait(idx_copies(t, s))
            wait(store_copies(t - _SLOTS, s))
            start(gather_copies(t, s))
            if t + 1 < n_chunks:
                start(idx_copies(t + 1, (t + 1) % _SLOTS))
            wait(gather_copies(t - 1, pj))
            start(store_copies(t - 1, pj))

        last = n_chunks - 1
        wait(gather_copies(last, last % _SLOTS))
        start(store_copies(last, last % _SLOTS))
        for u in range(n_chunks - _SLOTS, n_chunks):
            wait(store_copies(u, u % _SLOTS))

    pairs = gather_kernel(wpair, ih3)
    p3 = pairs.reshape(batch, seqp, 2 * dim)[:, :seq, :]
    odd = lax.bitwise_and(token_ids, 1)[..., None] == 1
    return jnp.where(odd, p3[..., dim:], p3[..., :dim])


# speedup vs baseline: 1.3019x; 1.3019x over previous
import functools

import jax
import jax.numpy as jnp
from jax import lax
from jax.experimental import pallas as pl
from jax.experimental.pallas import tpu as pltpu
from jax.experimental.pallas import tpu_sc as plsc

_NUM_CORES = 2
_NUM_SUBCORES = 16
_NUM_WORKERS = _NUM_CORES * _NUM_SUBCORES
_RPC = 2
_SLOTS = 3
_SENT = 2**30
_SEL_B = 64


def _select_body(x_ref, t_ref, o_ref):
    x = x_ref[...]
    t = jnp.swapaxes(t_ref[...], 1, 2)
    dim = o_ref.shape[-1]
    o_ref[...] = jnp.where(
        lax.bitwise_and(t, 1) == 1, x[..., dim:], x[..., :dim]
    )


def kernel(token_ids, weights):
    batch, seq = token_ids.shape
    num_rows, dim = weights.shape

    half = lax.shift_right_logical(token_ids, 1)
    halfp = jnp.pad(half, ((0, 0), (0, 128 - seq)), constant_values=_SENT)
    ih3 = halfp.reshape(batch // _RPC, _RPC, 128)
    wpair = weights.reshape(num_rows // 2, 2 * dim)

    rows_per_w = batch // _NUM_WORKERS
    n_chunks = rows_per_w // _RPC
    win = 128

    mesh = plsc.VectorSubcoreMesh(core_axis_name="c", subcore_axis_name="s")

    scratch = (
        [pltpu.VMEM((_RPC * win, 2 * dim), jnp.float32) for _ in range(_SLOTS)]
        + [pltpu.VMEM((_RPC, 128), jnp.int32) for _ in range(_SLOTS)]
        + [pltpu.SemaphoreType.DMA for _ in range(3 * _SLOTS)]
    )

    @functools.partial(
        pl.kernel,
        mesh=mesh,
        out_type=jax.ShapeDtypeStruct((batch, seq, 2 * dim), weights.dtype),
        scratch_types=scratch,
    )
    def gather_kernel(table_hbm, ih_hbm, out_hbm, *scr):
        rvs = scr[0:_SLOTS]
        ivs = scr[_SLOTS : 2 * _SLOTS]
        isem = scr[2 * _SLOTS : 3 * _SLOTS]
        gsem = scr[3 * _SLOTS : 4 * _SLOTS]
        ssem = scr[4 * _SLOTS : 5 * _SLOTS]

        wid = lax.axis_index("s") * _NUM_CORES + lax.axis_index("c")
        chunk_base = wid * n_chunks

        def idx_copies(t, s):
            return (
                pltpu.make_async_copy(ih_hbm.at[chunk_base + t], ivs[s], isem[s]),
            )

        def gather_copies(t, s):
            return tuple(
                pltpu.make_async_copy(
                    table_hbm.at[
                        plsc.Indices(ivs[s].at[r], ignored_value=_SENT)
                    ],
                    rvs[s].at[pl.ds(r * win, win)],
                    gsem[s],
                )
                for r in range(_RPC)
            )

        def store_copies(t, s):
            row0 = (chunk_base + t) * _RPC
            return tuple(
                pltpu.make_async_copy(
                    rvs[s].at[pl.ds(r * win, seq)], out_hbm.at[row0 + r], ssem[s]
                )
                for r in range(_RPC)
            )

        def start(cs):
            for c in cs:
                c.start()

        def wait(cs):
            for c in cs:
                c.wait()

        start(idx_copies(0, 0))
        for t in range(_SLOTS):
            s = t
            wait(idx_copies(t, s))
            start(gather_copies(t, s))
            start(idx_copies(t + 1, (t + 1) % _SLOTS))
            if t >= 1:
                wait(gather_copies(t - 1, s - 1))
                start(store_copies(t - 1, s - 1))

        @pl.loop(1, n_chunks // _SLOTS)
        def _(k):
            t0 = k * _SLOTS
            for j in range(_SLOTS):
                t = t0 + j
                s = j
                pj = (j - 1) % _SLOTS
                wait(idx_copies(t, s))
                wait(store_copies(t - _SLOTS, s))
                start(gather_copies(t, s))
                start(idx_copies(t + 1, (j + 1) % _SLOTS))
                wait(gather_copies(t - 1, pj))
                start(store_copies(t - 1, pj))

        for t in range(_SLOTS * (n_chunks // _SLOTS), n_chunks):
            s = t % _SLOTS
            pj = (s - 1) % _SLOTS
            wait(idx_copies(t, s))
            wait(store_copies(t - _SLOTS, s))
            start(gather_copies(t, s))
            if t + 1 < n_chunks:
                start(idx_copies(t + 1, (t + 1) % _SLOTS))
            wait(gather_copies(t - 1, pj))
            start(store_copies(t - 1, pj))

        last = n_chunks - 1
        wait(gather_copies(last, last % _SLOTS))
        start(store_copies(last, last % _SLOTS))
        for u in range(n_chunks - _SLOTS, n_chunks):
            wait(store_copies(u, u % _SLOTS))

    pairs = gather_kernel(wpair, ih3)

    tok3 = token_ids.reshape(batch, 1, seq)
    select = pl.pallas_call(
        _select_body,
        grid=(batch // _SEL_B,),
        in_specs=[
            pl.BlockSpec((_SEL_B, seq, 2 * dim), lambda i: (i, 0, 0)),
            pl.BlockSpec((_SEL_B, 1, seq), lambda i: (i, 0, 0)),
        ],
        out_specs=pl.BlockSpec((_SEL_B, seq, dim), lambda i: (i, 0, 0)),
        out_shape=jax.ShapeDtypeStruct((batch, seq, dim), weights.dtype),
    )
    return select(pairs, tok3)


# speedup vs baseline: 1.5018x; 1.1535x over previous
import functools

import jax
import jax.numpy as jnp
from jax import lax
from jax.experimental import pallas as pl
from jax.experimental.pallas import tpu as pltpu
from jax.experimental.pallas import tpu_sc as plsc

_NUM_CORES = 2
_NUM_SUBCORES = 16
_NUM_WORKERS = _NUM_CORES * _NUM_SUBCORES
_RPC = 4
_SLOTS = 4


def kernel(token_ids, weights):
    batch, seq = token_ids.shape
    num_idx = batch * seq
    num_rows, dim = weights.shape

    idx = token_ids.reshape(num_idx)
    half = lax.shift_right_logical(idx, 1)
    wpair = weights.reshape(num_rows // 2, 2 * dim)

    rows_per_w = batch // _NUM_WORKERS
    n_chunks = rows_per_w // _RPC
    chunk_idx = _RPC * seq
    na = 128
    nb = chunk_idx - na

    mesh = plsc.VectorSubcoreMesh(core_axis_name="c", subcore_axis_name="s")

    scratch = (
        [pltpu.VMEM((chunk_idx, 2 * dim), jnp.float32) for _ in range(_SLOTS)]
        + [pltpu.VMEM((na,), jnp.int32) for _ in range(_SLOTS)]
        + [pltpu.VMEM((nb,), jnp.int32) for _ in range(_SLOTS)]
        + [pltpu.SemaphoreType.DMA for _ in range(3 * _SLOTS)]
    )

    @functools.partial(
        pl.kernel,
        mesh=mesh,
        out_type=jax.ShapeDtypeStruct((batch, seq, 2 * dim), weights.dtype),
        scratch_types=scratch,
    )
    def gather_kernel(table_hbm, ih_hbm, out_hbm, *scr):
        rvs = scr[0:4]
        iva = scr[4:8]
        ivb = scr[8:12]
        isem = scr[12:16]
        gsem = scr[16:20]
        ssem = scr[20:24]

        wid = lax.axis_index("s") * _NUM_CORES + lax.axis_index("c")
        row_base = wid * rows_per_w

        def idx_copies(t, s):
            off = (row_base + t * _RPC) * seq
            return (
                pltpu.make_async_copy(ih_hbm.at[pl.ds(off, na)], iva[s], isem[s]),
                pltpu.make_async_copy(
                    ih_hbm.at[pl.ds(off + na, nb)], ivb[s], isem[s]
                ),
            )

        def gather_copies(t, s):
            return (
                pltpu.make_async_copy(
                    table_hbm.at[iva[s]], rvs[s].at[pl.ds(0, na)], gsem[s]
                ),
                pltpu.make_async_copy(
                    table_hbm.at[ivb[s]], rvs[s].at[pl.ds(na, nb)], gsem[s]
                ),
            )

        def store_copies(t, s):
            row0 = row_base + t * _RPC
            return tuple(
                pltpu.make_async_copy(
                    rvs[s].at[pl.ds(r * seq, seq)], out_hbm.at[row0 + r], ssem[s]
                )
                for r in range(_RPC)
            )

        def start(cs):
            for c in cs:
                c.start()

        def wait(cs):
            for c in cs:
                c.wait()

        start(idx_copies(0, 0))
        for t in range(_SLOTS):
            s = t
            wait(idx_copies(t, s))
            start(gather_copies(t, s))
            start(idx_copies(t + 1, (t + 1) % _SLOTS))
            if t >= 1:
                wait(gather_copies(t - 1, s - 1))
                start(store_copies(t - 1, s - 1))

        @pl.loop(1, n_chunks // _SLOTS - 1)
        def _(k):
            t0 = k * _SLOTS
            for j in range(_SLOTS):
                t = t0 + j
                s = j
                pj = (j - 1) % _SLOTS
                wait(idx_copies(t, s))
                wait(store_copies(t - _SLOTS, s))
                start(gather_copies(t, s))
                start(idx_copies(t + 1, (j + 1) % _SLOTS))
                wait(gather_copies(t - 1, pj))
                start(store_copies(t - 1, pj))

        for j in range(_SLOTS):
            t = n_chunks - _SLOTS + j
            s = j
            pj = (j - 1) % _SLOTS
            wait(idx_copies(t, s))
            wait(store_copies(t - _SLOTS, s))
            start(gather_copies(t, s))
            if t + 1 < n_chunks:
                start(idx_copies(t + 1, (j + 1) % _SLOTS))
            wait(gather_copies(t - 1, pj))
            start(store_copies(t - 1, pj))

        wait(gather_copies(n_chunks - 1, _SLOTS - 1))
        start(store_copies(n_chunks - 1, _SLOTS - 1))
        for j in range(_SLOTS):
            wait(store_copies(n_chunks - _SLOTS + j, j))

    pairs = gather_kernel(wpair, half)
    odd = lax.bitwise_and(token_ids, 1)[..., None] == 1
    return jnp.where(odd, pairs[..., dim:], pairs[..., :dim])
